# baseline (speedup 1.0000x reference)
import functools

import jax
import jax.numpy as jnp
from jax import lax
from jax.experimental import pallas as pl
from jax.experimental.pallas import tpu as pltpu
from jax.experimental.pallas import tpu_sc as plsc

_N = 10000
_D = 128
_E = 320000
_EN = _E + _N
_AR = 10008
_CH = 112
_NSC = 2
_NTS = 16
_NW = _NSC * _NTS
_CHUNKS = 93
_EPT = _CHUNKS * _CH
_EP = _NW * _EPT
_K0 = 60
_K1 = 2 * _CHUNKS - _K0
_EPT0 = _K0 * _CH
_EPT1 = _K1 * _CH
_RPT = 624
_G16 = _CH // 16
_V16 = _D // 16
_NPD = 10016

_MESH = plsc.VectorSubcoreMesh(
    core_axis_name="c", subcore_axis_name="s", num_cores=_NSC, num_subcores=_NTS
)
_SC_PARAMS = pltpu.CompilerParams(needs_layout_passes=False)


def _tc_prep_body(x_ref, w_ref, as_ref, ad_ref, xp_ref, av_ref, bv_ref, m_ref):
    xp = jnp.dot(x_ref[...], w_ref[...], preferred_element_type=jnp.float32)
    a_s = jnp.sum(xp * as_ref[...], axis=1, keepdims=True)
    a_d = jnp.sum(xp * ad_ref[...], axis=1, keepdims=True)
    av_ref[...] = a_s
    bv_ref[...] = a_d
    mx = jnp.max(a_s) + jnp.max(a_d)
    mx = jnp.where(mx < 0.0, mx * jnp.float32(0.2), mx)
    m_ref[...] = jnp.full((1, 128), mx, jnp.float32)
    xp_ref[...] = xp


_tc_prep = pl.pallas_call(
    _tc_prep_body,
    out_shape=[
        jax.ShapeDtypeStruct((_N, _D), jnp.float32),
        jax.ShapeDtypeStruct((_N, 1), jnp.float32),
        jax.ShapeDtypeStruct((_N, 1), jnp.float32),
        jax.ShapeDtypeStruct((1, 128), jnp.float32),
    ],
)


@functools.partial(
    pl.kernel,
    out_type=[
        jax.ShapeDtypeStruct((_NSC, _AR, _D), jnp.float32),
        jax.ShapeDtypeStruct((_AR,), jnp.float32),
        jax.ShapeDtypeStruct((_AR,), jnp.float32),
        jax.ShapeDtypeStruct((_EP,), jnp.float32),
    ],
    mesh=_MESH,
    scratch_types=[
        pltpu.VMEM((16,), jnp.float32),
        pltpu.VMEM((3, _CH), jnp.int32),
        pltpu.VMEM((3, _CH), jnp.int32),
        pltpu.VMEM((3, _CH), jnp.int32),
        pltpu.VMEM((3, _CH), jnp.float32),
        pltpu.VMEM((3, _CH), jnp.float32),
        pltpu.VMEM((3, _CH), jnp.float32),
        pltpu.VMEM((_CH, _D), jnp.float32),
        pltpu.VMEM((_CH, _D), jnp.float32),
        pltpu.VMEM((_CH, _D), jnp.float32),
        pltpu.VMEM_SHARED((_AR, _D), jnp.float32),
        pltpu.VMEM_SHARED((_AR,), jnp.float32),
        pltpu.SemaphoreType.DMA,
        pltpu.SemaphoreType.DMA,
        pltpu.SemaphoreType.DMA,
        pltpu.SemaphoreType.DMA,
        pltpu.SemaphoreType.DMA,
        pltpu.SemaphoreType.DMA,
        pltpu.SemaphoreType.DMA,
        pltpu.SemaphoreType.DMA,
        pltpu.SemaphoreType.DMA,
        pltpu.SemaphoreType.DMA,
        pltpu.SemaphoreType.DMA,
        pltpu.SemaphoreType.DMA,
        pltpu.SemaphoreType.DMA,
        pltpu.SemaphoreType.DMA,
        pltpu.SemaphoreType.DMA,
        pltpu.SemaphoreType.DMA,
    ],
    compiler_params=_SC_PARAMS,
)
def _sc_edges(xp_hbm, src_hbm, dst_hbm, as_hbm, ad_hbm, m_hbm,
              acc_out, accd_out0, accd_out1, p_out,
              m_v, srcv, dstv, dscv, asg, adg, pv,
              rows0_v, rows1_v, rows2_v, acc_sh, accd_sh,
              lsem, isem0, isem1, isem2, gsem0, gsem1, gsem2,
              ssem0, ssem1, ssem2, dsem0, dsem1, dsem2,
              psem0, psem1, psem2):
    cid = lax.axis_index("c")
    sid = lax.axis_index("s")
    wid = cid * _NTS + sid
    rows = (rows0_v, rows1_v, rows2_v)
    isem = (isem0, isem1, isem2)
    gsem = (gsem0, gsem1, gsem2)
    ssem = (ssem0, ssem1, ssem2)
    dsem = (dsem0, dsem1, dsem2)
    psem = (psem0, psem1, psem2)

    ld_m = pltpu.async_copy(m_hbm, m_v, lsem)

    def _zrow(r, carry):
        for j in range(_V16):
            rows0_v[r, pl.ds(j * 16, 16)] = jnp.zeros((16,), jnp.float32)
        return carry

    lax.fori_loop(0, _CH, _zrow, 0)
    for g in range(_G16):
        pv[0, pl.ds(g * 16, 16)] = jnp.zeros((16,), jnp.float32)

    base_r = sid * _RPT
    _zchunks = [(0, _CH), (_CH, _CH), (2 * _CH, _CH), (3 * _CH, _CH),
                (4 * _CH, _CH), (5 * _CH, _RPT - 5 * _CH)]

    def _zero_range(r0, chunks):
        for off, ln in chunks:
            pltpu.sync_copy(rows0_v.at[pl.ds(0, ln)],
                            acc_sh.at[pl.ds(r0 + off, ln)])
            pltpu.sync_copy(pv.at[0, pl.ds(0, ln)],
                            accd_sh.at[pl.ds(r0 + off, ln)])

    _zero_range(base_r, _zchunks)

    @pl.when(sid == _NTS - 1)
    def _():
        _zero_range(_NTS * _RPT, [(0, _AR - _NTS * _RPT)])

    plsc.subcore_barrier()
    ld_m.wait()
    mvec = m_v[...]

    kk = jnp.where(cid == 0, jnp.int32(_K0), jnp.int32(_K1))
    ebase = jnp.where(cid == 0, sid * _EPT0,
                      _NTS * _EPT0 + sid * _EPT1)

    def _load_idx(c, b):
        pltpu.async_copy(src_hbm.at[pl.ds(ebase + c * _CH, _CH)], srcv.at[b],
                         isem[b])
        pltpu.async_copy(dst_hbm.at[pl.ds(ebase + c * _CH, _CH)], dstv.at[b],
                         isem[b])

    def _wait_idx(c, b):
        pltpu.make_async_copy(src_hbm.at[pl.ds(0, _CH)], srcv.at[b],
                              isem[b]).wait()
        pltpu.make_async_copy(dst_hbm.at[pl.ds(0, _CH)], dstv.at[b],
                              isem[b]).wait()

    def _start_gather(b):
        pltpu.async_copy(xp_hbm.at[srcv.at[b]], rows[b], gsem[b])
        pltpu.async_copy(as_hbm.at[srcv.at[b]], asg.at[b], gsem[b])
        pltpu.async_copy(ad_hbm.at[dstv.at[b]], adg.at[b], gsem[b])

    def _wait_gather(b):
        pltpu.make_async_copy(xp_hbm.at[pl.ds(0, _CH)], rows[b],
                              gsem[b]).wait()
        pltpu.make_async_copy(as_hbm.at[pl.ds(0, _CH)], asg.at[b],
                              gsem[b]).wait()
        pltpu.make_async_copy(ad_hbm.at[pl.ds(0, _CH)], adg.at[b],
                              gsem[b]).wait()

    def _start_scatter(c, b):
        pltpu.async_copy(rows[b], acc_sh.at[dscv.at[b]], ssem[b], add=True)
        pltpu.async_copy(pv.at[b], accd_sh.at[dscv.at[b]], dsem[b], add=True)
        pltpu.async_copy(pv.at[b], p_out.at[pl.ds(ebase + c * _CH, _CH)],
                         psem[b])

    def _wait_scatter(c, b):
        pltpu.make_async_copy(xp_hbm.at[pl.ds(0, _CH)], rows[b],
                              ssem[b]).wait()
        pltpu.make_async_copy(as_hbm.at[pl.ds(0, _CH)], pv.at[b],
                              dsem[b]).wait()
        pltpu.make_async_copy(pv.at[b], p_out.at[pl.ds(0, _CH)],
                              psem[b]).wait()

    def _compute(c, b):
        for g in range(_G16):
            sl = pl.ds(g * 16, 16)
            e = asg[b, sl] + adg[b, sl]
            e = jnp.where(e < 0.0, e * jnp.float32(0.2), e)
            pv[b, sl] = jnp.exp(e - mvec)
            dscv[b, sl] = dstv[b, sl]

        def _edge(r, c2):
            rb = jnp.broadcast_to(r, (16,)).astype(jnp.int32)
            pb = plsc.load_gather(pv.at[b], [rb])
            for j in range(_V16):
                sl2 = pl.ds(j * 16, 16)
                rows[b][r, sl2] = rows[b][r, sl2] * pb
            return c2

        lax.fori_loop(0, _CH, _edge, 0, unroll=4)

    def _slot(c, b, wait_prev=True, load_next2=True, start_next=True):
        bn = (b + 1) % 3
        bnn = (b + 2) % 3
        if wait_prev:
            _wait_scatter(c - 2, bn)
        if start_next:
            _wait_idx(c + 1, bn)
            _start_gather(bn)
        if load_next2:
            _load_idx(c + 2, bnn)
        _wait_gather(b)
        _compute(c, b)
        _start_scatter(c, b)

    _load_idx(jnp.int32(0), 0)
    _wait_idx(jnp.int32(0), 0)
    _start_gather(0)
    _load_idx(jnp.int32(1), 1)
    _slot(jnp.int32(0), 0, wait_prev=False)
    _slot(jnp.int32(1), 1, wait_prev=False)

    def _triple(j, carry):
        c = 3 * j + 2
        _slot(c, 2)
        _slot(c + 1, 0)
        _slot(c + 2, 1)
        return carry

    lax.fori_loop(0, (kk - 4) // 3, _triple, 0)
    _slot(kk - 4, 2)
    _slot(kk - 3, 0)
    _slot(kk - 2, 1, load_next2=False)
    _slot(kk - 1, 2, load_next2=False, start_next=False)
    _wait_scatter(kk - 2, 1)
    _wait_scatter(kk - 1, 2)
    plsc.subcore_barrier()

    def _copy_range(r0, chunks, accd_dst):
        for off, ln in chunks:
            pltpu.sync_copy(acc_sh.at[pl.ds(r0 + off, ln)],
                            rows0_v.at[pl.ds(0, ln)])
            pltpu.sync_copy(rows0_v.at[pl.ds(0, ln)],
                            acc_out.at[cid, pl.ds(r0 + off, ln)])
            pltpu.sync_copy(accd_sh.at[pl.ds(r0 + off, ln)],
                            pv.at[0, pl.ds(0, ln)])
            pltpu.sync_copy(pv.at[0, pl.ds(0, ln)],
                            accd_dst.at[pl.ds(r0 + off, ln)])

    @pl.when(cid == 0)
    def _():
        _copy_range(base_r, _zchunks, accd_out0)

        @pl.when(sid == _NTS - 1)
        def _():
            _copy_range(_NTS * _RPT, [(0, _AR - _NTS * _RPT)], accd_out0)

    @pl.when(cid == 1)
    def _():
        _copy_range(base_r, _zchunks, accd_out1)

        @pl.when(sid == _NTS - 1)
        def _():
            _copy_range(_NTS * _RPT, [(0, _AR - _NTS * _RPT)], accd_out1)


def _tc_finish_body(acc_ref, accd_ref, b_ref, h_ref):
    s = acc_ref[0] + acc_ref[1]
    den = accd_ref[0] + accd_ref[1]
    rr = 1.0 / (den + jnp.float32(1e-16))
    h_ref[...] = s * rr + b_ref[...]


_tc_finish = pl.pallas_call(
    _tc_finish_body,
    out_shape=jax.ShapeDtypeStruct((_AR, _D), jnp.float32),
)


@functools.partial(
    pl.kernel,
    out_type=jax.ShapeDtypeStruct((_EP,), jnp.float32),
    mesh=_MESH,
    scratch_types=[
        pltpu.VMEM((_NPD,), jnp.float32),
        pltpu.VMEM((_NPD,), jnp.float32),
        pltpu.VMEM((_EPT,), jnp.float32),
        pltpu.VMEM((_EPT,), jnp.int32),
        pltpu.VMEM((_EPT,), jnp.float32),
        pltpu.SemaphoreType.DMA,
    ],
    compiler_params=_SC_PARAMS,
)
def _sc_alpha(p_hbm, dst_hbm, accd0_hbm, accd1_hbm, alpha_out,
              r_v, d1_v, pall_v, dstall_v, aall_v, lsem):
    cid = lax.axis_index("c")
    sid = lax.axis_index("s")
    wid = cid * _NTS + sid
    ebase = wid * _EPT
    lds = [
        pltpu.async_copy(accd0_hbm, r_v.at[pl.ds(0, _AR)], lsem),
        pltpu.async_copy(accd1_hbm, d1_v.at[pl.ds(0, _AR)], lsem),
        pltpu.async_copy(p_hbm.at[pl.ds(ebase, _EPT)], pall_v, lsem),
        pltpu.async_copy(dst_hbm.at[pl.ds(ebase, _EPT)], dstall_v, lsem),
    ]
    for h in lds:
        h.wait()

    def _rinit(i, carry):
        sl = pl.ds(i * 16, 16)
        r_v[sl] = 1.0 / (r_v[sl] + d1_v[sl] + jnp.float32(1e-16))
        return carry

    lax.fori_loop(0, _NPD // 16, _rinit, 0)

    def _grp(i, carry):
        sl = pl.ds(i * 16, 16)
        rv = plsc.load_gather(r_v, [dstall_v[sl]])
        aall_v[sl] = pall_v[sl] * rv
        return carry

    lax.fori_loop(0, _EPT // 16, _grp, 0, unroll=4)
    pltpu.sync_copy(aall_v, alpha_out.at[pl.ds(ebase, _EPT)])


def kernel(x, edge_index, W, att_src, att_dst, bias):
    xp, a_s, a_d, m = _tc_prep(
        x, W, att_src.reshape(1, _D), att_dst.reshape(1, _D)
    )
    a_s = a_s.reshape(_N)
    a_d = jnp.concatenate([a_d.reshape(_N), jnp.zeros((_NPD - _N,), jnp.float32)])
    m16 = m.reshape(128)[:16]

    loop = jnp.arange(_N, dtype=jnp.int32)
    npad_e = _EP - _EN
    src_full = jnp.concatenate(
        [edge_index[0], loop, jnp.zeros((npad_e,), jnp.int32)])
    dst_full = jnp.concatenate(
        [edge_index[1], loop, jnp.full((npad_e,), _N, jnp.int32)])

    acc, accd0, accd1, p = _sc_edges(xp, src_full, dst_full, a_s, a_d, m16)
    accd = jnp.stack([accd0, accd1]).reshape(_NSC, _AR, 1)
    h_full = _tc_finish(acc, accd, bias.reshape(1, _D))
    alpha = _sc_alpha(p, dst_full, accd0, accd1)
    return h_full[:_N], alpha[:_EN].reshape(_EN, 1)


# speedup vs baseline: 1.0146x; 1.0146x over previous
import functools

import jax
import jax.numpy as jnp
from jax import lax
from jax.experimental import pallas as pl
from jax.experimental.pallas import tpu as pltpu
from jax.experimental.pallas import tpu_sc as plsc

_N = 10000
_D = 128
_E = 320000
_EN = _E + _N
_AR = 10008
_CH = 112
_NSC = 2
_NTS = 16
_NW = _NSC * _NTS
_CHUNKS = 93
_EPT = _CHUNKS * _CH
_EP = _NW * _EPT
_K0 = 60
_K1 = 2 * _CHUNKS - _K0
_EPT0 = _K0 * _CH
_EPT1 = _K1 * _CH
_RPT = 624
_G16 = _CH // 16
_V16 = _D // 16
_NPD = 10016

_MESH = plsc.VectorSubcoreMesh(
    core_axis_name="c", subcore_axis_name="s", num_cores=_NSC, num_subcores=_NTS
)
_SC_PARAMS = pltpu.CompilerParams(needs_layout_passes=False)


def _tc_prep_body(x_ref, w_ref, as_ref, ad_ref, xp_ref, av_ref, bv_ref, m_ref):
    xp = jnp.dot(x_ref[...], w_ref[...], preferred_element_type=jnp.float32)
    a_s = jnp.sum(xp * as_ref[...], axis=1, keepdims=True)
    a_d = jnp.sum(xp * ad_ref[...], axis=1, keepdims=True)
    av_ref[...] = a_s
    bv_ref[...] = a_d
    mx = jnp.max(a_s) + jnp.max(a_d)
    mx = jnp.where(mx < 0.0, mx * jnp.float32(0.2), mx)
    m_ref[...] = jnp.full((1, 128), mx, jnp.float32)
    xp_ref[...] = xp


_tc_prep = pl.pallas_call(
    _tc_prep_body,
    out_shape=[
        jax.ShapeDtypeStruct((_N, _D), jnp.float32),
        jax.ShapeDtypeStruct((_N, 1), jnp.float32),
        jax.ShapeDtypeStruct((_N, 1), jnp.float32),
        jax.ShapeDtypeStruct((1, 128), jnp.float32),
    ],
)


@functools.partial(
    pl.kernel,
    out_type=[
        jax.ShapeDtypeStruct((_NSC, _AR, _D), jnp.float32),
        jax.ShapeDtypeStruct((_AR,), jnp.float32),
        jax.ShapeDtypeStruct((_AR,), jnp.float32),
        jax.ShapeDtypeStruct((_EP,), jnp.float32),
    ],
    mesh=_MESH,
    scratch_types=[
        pltpu.VMEM((16,), jnp.float32),
        pltpu.VMEM((3, _CH), jnp.int32),
        pltpu.VMEM((3, _CH), jnp.int32),
        pltpu.VMEM((3, _CH), jnp.int32),
        pltpu.VMEM((3, _CH), jnp.float32),
        pltpu.VMEM((3, _CH), jnp.float32),
        pltpu.VMEM((3, _CH), jnp.float32),
        pltpu.VMEM((_CH, _D), jnp.float32),
        pltpu.VMEM((_CH, _D), jnp.float32),
        pltpu.VMEM((_CH, _D), jnp.float32),
        pltpu.VMEM_SHARED((_AR, _D), jnp.float32),
        pltpu.VMEM_SHARED((_AR,), jnp.float32),
        pltpu.SemaphoreType.DMA,
        pltpu.SemaphoreType.DMA,
        pltpu.SemaphoreType.DMA,
        pltpu.SemaphoreType.DMA,
        pltpu.SemaphoreType.DMA,
        pltpu.SemaphoreType.DMA,
        pltpu.SemaphoreType.DMA,
        pltpu.SemaphoreType.DMA,
        pltpu.SemaphoreType.DMA,
        pltpu.SemaphoreType.DMA,
        pltpu.SemaphoreType.DMA,
        pltpu.SemaphoreType.DMA,
        pltpu.SemaphoreType.DMA,
        pltpu.SemaphoreType.DMA,
        pltpu.SemaphoreType.DMA,
        pltpu.SemaphoreType.DMA,
    ],
    compiler_params=_SC_PARAMS,
)
def _sc_edges(xp_hbm, src_hbm, dst_hbm, as_hbm, ad_hbm, m_hbm,
              acc_out, accd_out0, accd_out1, p_out,
              m_v, srcv, dstv, dscv, asg, adg, pv,
              rows0_v, rows1_v, rows2_v, acc_sh, accd_sh,
              lsem, isem0, isem1, isem2, gsem0, gsem1, gsem2,
              ssem0, ssem1, ssem2, dsem0, dsem1, dsem2,
              psem0, psem1, psem2):
    cid = lax.axis_index("c")
    sid = lax.axis_index("s")
    wid = cid * _NTS + sid
    rows = (rows0_v, rows1_v, rows2_v)
    isem = (isem0, isem1, isem2)
    gsem = (gsem0, gsem1, gsem2)
    ssem = (ssem0, ssem1, ssem2)
    dsem = (dsem0, dsem1, dsem2)
    psem = (psem0, psem1, psem2)

    ld_m = pltpu.async_copy(m_hbm, m_v, lsem)

    def _zrow(r, carry):
        for j in range(_V16):
            rows0_v[r, pl.ds(j * 16, 16)] = jnp.zeros((16,), jnp.float32)
        return carry

    lax.fori_loop(0, _CH, _zrow, 0)
    for g in range(_G16):
        pv[0, pl.ds(g * 16, 16)] = jnp.zeros((16,), jnp.float32)

    base_r = sid * _RPT
    _zchunks = [(0, _CH), (_CH, _CH), (2 * _CH, _CH), (3 * _CH, _CH),
                (4 * _CH, _CH), (5 * _CH, _RPT - 5 * _CH)]

    def _zero_range(r0, chunks):
        for off, ln in chunks:
            pltpu.sync_copy(rows0_v.at[pl.ds(0, ln)],
                            acc_sh.at[pl.ds(r0 + off, ln)])
            pltpu.sync_copy(pv.at[0, pl.ds(0, ln)],
                            accd_sh.at[pl.ds(r0 + off, ln)])

    _zero_range(base_r, _zchunks)

    @pl.when(sid == _NTS - 1)
    def _():
        _zero_range(_NTS * _RPT, [(0, _AR - _NTS * _RPT)])

    plsc.subcore_barrier()
    ld_m.wait()
    mvec = m_v[...]

    kk = jnp.where(cid == 1, jnp.int32(_K0), jnp.int32(_K1))
    ebase = jnp.where(cid == 1, sid * _EPT0,
                      _NTS * _EPT0 + sid * _EPT1)

    def _load_idx(c, b):
        pltpu.async_copy(src_hbm.at[pl.ds(ebase + c * _CH, _CH)], srcv.at[b],
                         isem[b])
        pltpu.async_copy(dst_hbm.at[pl.ds(ebase + c * _CH, _CH)], dstv.at[b],
                         isem[b])

    def _wait_idx(c, b):
        pltpu.make_async_copy(src_hbm.at[pl.ds(0, _CH)], srcv.at[b],
                              isem[b]).wait()
        pltpu.make_async_copy(dst_hbm.at[pl.ds(0, _CH)], dstv.at[b],
                              isem[b]).wait()

    def _start_gather(b):
        pltpu.async_copy(xp_hbm.at[srcv.at[b]], rows[b], gsem[b])
        pltpu.async_copy(as_hbm.at[srcv.at[b]], asg.at[b], gsem[b])
        pltpu.async_copy(ad_hbm.at[dstv.at[b]], adg.at[b], gsem[b])

    def _wait_gather(b):
        pltpu.make_async_copy(xp_hbm.at[pl.ds(0, _CH)], rows[b],
                              gsem[b]).wait()
        pltpu.make_async_copy(as_hbm.at[pl.ds(0, _CH)], asg.at[b],
                              gsem[b]).wait()
        pltpu.make_async_copy(ad_hbm.at[pl.ds(0, _CH)], adg.at[b],
                              gsem[b]).wait()

    def _start_scatter(c, b):
        pltpu.async_copy(rows[b], acc_sh.at[dscv.at[b]], ssem[b], add=True)
        pltpu.async_copy(pv.at[b], accd_sh.at[dscv.at[b]], dsem[b], add=True)
        pltpu.async_copy(pv.at[b], p_out.at[pl.ds(ebase + c * _CH, _CH)],
                         psem[b])

    def _wait_scatter(c, b):
        pltpu.make_async_copy(xp_hbm.at[pl.ds(0, _CH)], rows[b],
                              ssem[b]).wait()
        pltpu.make_async_copy(as_hbm.at[pl.ds(0, _CH)], pv.at[b],
                              dsem[b]).wait()
        pltpu.make_async_copy(pv.at[b], p_out.at[pl.ds(0, _CH)],
                              psem[b]).wait()

    def _compute(c, b):
        for g in range(_G16):
            sl = pl.ds(g * 16, 16)
            e = asg[b, sl] + adg[b, sl]
            e = jnp.where(e < 0.0, e * jnp.float32(0.2), e)
            pv[b, sl] = jnp.exp(e - mvec)
            dscv[b, sl] = dstv[b, sl]

        def _edge(r, c2):
            rb = jnp.broadcast_to(r, (16,)).astype(jnp.int32)
            pb = plsc.load_gather(pv.at[b], [rb])
            for j in range(_V16):
                sl2 = pl.ds(j * 16, 16)
                rows[b][r, sl2] = rows[b][r, sl2] * pb
            return c2

        lax.fori_loop(0, _CH, _edge, 0, unroll=4)

    def _slot(c, b, wait_prev=True, load_next2=True, start_next=True):
        bn = (b + 1) % 3
        bnn = (b + 2) % 3
        if wait_prev:
            _wait_scatter(c - 2, bn)
        if start_next:
            _wait_idx(c + 1, bn)
            _start_gather(bn)
        if load_next2:
            _load_idx(c + 2, bnn)
        _wait_gather(b)
        _compute(c, b)
        _start_scatter(c, b)

    _load_idx(jnp.int32(0), 0)
    _wait_idx(jnp.int32(0), 0)
    _start_gather(0)
    _load_idx(jnp.int32(1), 1)
    _slot(jnp.int32(0), 0, wait_prev=False)
    _slot(jnp.int32(1), 1, wait_prev=False)

    def _triple(j, carry):
        c = 3 * j + 2
        _slot(c, 2)
        _slot(c + 1, 0)
        _slot(c + 2, 1)
        return carry

    lax.fori_loop(0, (kk - 4) // 3, _triple, 0)
    _slot(kk - 4, 2)
    _slot(kk - 3, 0)
    _slot(kk - 2, 1, load_next2=False)
    _slot(kk - 1, 2, load_next2=False, start_next=False)
    _wait_scatter(kk - 2, 1)
    _wait_scatter(kk - 1, 2)
    plsc.subcore_barrier()

    def _copy_range(r0, chunks, accd_dst):
        for off, ln in chunks:
            pltpu.sync_copy(acc_sh.at[pl.ds(r0 + off, ln)],
                            rows0_v.at[pl.ds(0, ln)])
            pltpu.sync_copy(rows0_v.at[pl.ds(0, ln)],
                            acc_out.at[cid, pl.ds(r0 + off, ln)])
            pltpu.sync_copy(accd_sh.at[pl.ds(r0 + off, ln)],
                            pv.at[0, pl.ds(0, ln)])
            pltpu.sync_copy(pv.at[0, pl.ds(0, ln)],
                            accd_dst.at[pl.ds(r0 + off, ln)])

    @pl.when(cid == 0)
    def _():
        _copy_range(base_r, _zchunks, accd_out0)

        @pl.when(sid == _NTS - 1)
        def _():
            _copy_range(_NTS * _RPT, [(0, _AR - _NTS * _RPT)], accd_out0)

    @pl.when(cid == 1)
    def _():
        _copy_range(base_r, _zchunks, accd_out1)

        @pl.when(sid == _NTS - 1)
        def _():
            _copy_range(_NTS * _RPT, [(0, _AR - _NTS * _RPT)], accd_out1)


def _tc_finish_body(acc_ref, accd_ref, b_ref, h_ref):
    s = acc_ref[0] + acc_ref[1]
    den = accd_ref[0] + accd_ref[1]
    rr = 1.0 / (den + jnp.float32(1e-16))
    h_ref[...] = s * rr + b_ref[...]


_tc_finish = pl.pallas_call(
    _tc_finish_body,
    out_shape=jax.ShapeDtypeStruct((_AR, _D), jnp.float32),
)


@functools.partial(
    pl.kernel,
    out_type=jax.ShapeDtypeStruct((_EP,), jnp.float32),
    mesh=_MESH,
    scratch_types=[
        pltpu.VMEM((_NPD,), jnp.float32),
        pltpu.VMEM((_NPD,), jnp.float32),
        pltpu.VMEM((_EPT,), jnp.float32),
        pltpu.VMEM((_EPT,), jnp.int32),
        pltpu.VMEM((_EPT,), jnp.float32),
        pltpu.SemaphoreType.DMA,
    ],
    compiler_params=_SC_PARAMS,
)
def _sc_alpha(p_hbm, dst_hbm, accd0_hbm, accd1_hbm, alpha_out,
              r_v, d1_v, pall_v, dstall_v, aall_v, lsem):
    cid = lax.axis_index("c")
    sid = lax.axis_index("s")
    wid = cid * _NTS + sid
    ebase = wid * _EPT
    lds = [
        pltpu.async_copy(accd0_hbm, r_v.at[pl.ds(0, _AR)], lsem),
        pltpu.async_copy(accd1_hbm, d1_v.at[pl.ds(0, _AR)], lsem),
        pltpu.async_copy(p_hbm.at[pl.ds(ebase, _EPT)], pall_v, lsem),
        pltpu.async_copy(dst_hbm.at[pl.ds(ebase, _EPT)], dstall_v, lsem),
    ]
    for h in lds:
        h.wait()

    def _rinit(i, carry):
        sl = pl.ds(i * 16, 16)
        r_v[sl] = 1.0 / (r_v[sl] + d1_v[sl] + jnp.float32(1e-16))
        return carry

    lax.fori_loop(0, _NPD // 16, _rinit, 0)

    def _grp(i, carry):
        sl = pl.ds(i * 16, 16)
        rv = plsc.load_gather(r_v, [dstall_v[sl]])
        aall_v[sl] = pall_v[sl] * rv
        return carry

    lax.fori_loop(0, _EPT // 16, _grp, 0, unroll=4)
    pltpu.sync_copy(aall_v, alpha_out.at[pl.ds(ebase, _EPT)])


def kernel(x, edge_index, W, att_src, att_dst, bias):
    xp, a_s, a_d, m = _tc_prep(
        x, W, att_src.reshape(1, _D), att_dst.reshape(1, _D)
    )
    a_s = a_s.reshape(_N)
    a_d = jnp.concatenate([a_d.reshape(_N), jnp.zeros((_NPD - _N,), jnp.float32)])
    m16 = m.reshape(128)[:16]

    loop = jnp.arange(_N, dtype=jnp.int32)
    npad_e = _EP - _EN
    src_full = jnp.concatenate(
        [edge_index[0], loop, jnp.zeros((npad_e,), jnp.int32)])
    dst_full = jnp.concatenate(
        [edge_index[1], loop, jnp.full((npad_e,), _N, jnp.int32)])

    acc, accd0, accd1, p = _sc_edges(xp, src_full, dst_full, a_s, a_d, m16)
    accd = jnp.stack([accd0, accd1]).reshape(_NSC, _AR, 1)
    h_full = _tc_finish(acc, accd, bias.reshape(1, _D))
    alpha = _sc_alpha(p, dst_full, accd0, accd1)
    return h_full[:_N], alpha[:_EN].reshape(_EN, 1)


# speedup vs baseline: 1.1516x; 1.1351x over previous
import functools

import jax
import jax.numpy as jnp
from jax import lax
from jax.experimental import pallas as pl
from jax.experimental.pallas import tpu as pltpu
from jax.experimental.pallas import tpu_sc as plsc

_N = 10000
_D = 128
_Q = _D // 2
_E = 320000
_EN = _E + _N
_AR = 10008
_CH = 112
_NSC = 2
_NTS = 16
_NW = _NSC * _NTS
_CHUNKS = 93
_EPT = _CHUNKS * _CH
_EP = _NW * _EPT
_RPT = 624
_G16 = _CH // 16
_NPD = 10016

_MESH = plsc.VectorSubcoreMesh(
    core_axis_name="c", subcore_axis_name="s", num_cores=_NSC, num_subcores=_NTS
)
_SC_PARAMS = pltpu.CompilerParams(needs_layout_passes=False)


def _tc_prep_body(x_ref, w_ref, as_ref, ad_ref, xp_ref, av_ref, bv_ref, m_ref):
    xp = jnp.dot(x_ref[...], w_ref[...], preferred_element_type=jnp.float32)
    a_s = jnp.sum(xp * as_ref[...], axis=1, keepdims=True)
    a_d = jnp.sum(xp * ad_ref[...], axis=1, keepdims=True)
    av_ref[...] = a_s
    bv_ref[...] = a_d
    mx = jnp.max(a_s) + jnp.max(a_d)
    mx = jnp.where(mx < 0.0, mx * jnp.float32(0.2), mx)
    m_ref[...] = jnp.full((1, 128), mx, jnp.float32)
    xp_ref[...] = xp


_tc_prep = pl.pallas_call(
    _tc_prep_body,
    out_shape=[
        jax.ShapeDtypeStruct((_N, _D), jnp.float32),
        jax.ShapeDtypeStruct((_N, 1), jnp.float32),
        jax.ShapeDtypeStruct((_N, 1), jnp.float32),
        jax.ShapeDtypeStruct((1, 128), jnp.float32),
    ],
)


@functools.partial(
    pl.kernel,
    out_type=[
        jax.ShapeDtypeStruct((_NSC, _AR, _D), jnp.float32),
        jax.ShapeDtypeStruct((_AR,), jnp.float32),
        jax.ShapeDtypeStruct((_AR,), jnp.float32),
        jax.ShapeDtypeStruct((_EP,), jnp.float32),
    ],
    mesh=_MESH,
    scratch_types=[
        pltpu.VMEM((16,), jnp.float32),
        pltpu.VMEM((2, _CH), jnp.int32),
        pltpu.VMEM((2, _CH), jnp.int32),
        pltpu.VMEM((2, _CH), jnp.int32),
        pltpu.VMEM((2, _CH), jnp.float32),
        pltpu.VMEM((2, _CH), jnp.float32),
        pltpu.VMEM((2, _CH), jnp.float32),
        pltpu.VMEM((_CH, _D), jnp.float32),
        pltpu.VMEM((_CH, _D), jnp.float32),
        pltpu.VMEM_SHARED((_AR, _D), jnp.float32),
        pltpu.VMEM_SHARED((_AR,), jnp.float32),
        pltpu.SemaphoreType.DMA,
        pltpu.SemaphoreType.DMA,
        pltpu.SemaphoreType.DMA,
        pltpu.SemaphoreType.DMA,
        pltpu.SemaphoreType.DMA,
        pltpu.SemaphoreType.DMA,
        pltpu.SemaphoreType.DMA,
        pltpu.SemaphoreType.DMA,
        pltpu.SemaphoreType.DMA,
        pltpu.SemaphoreType.DMA,
        pltpu.SemaphoreType.DMA,
    ],
    compiler_params=_SC_PARAMS,
)
def _sc_edges(xp_hbm, src_hbm, dst_hbm, as_hbm, ad_hbm, m_hbm,
              acc_out, accd_out0, accd_out1, p_out,
              m_v, srcv, dstv, dscv, asg, adg, pv,
              fb0, fb1, acc_sh, accd_sh,
              lsem, isem0, isem1, gsem0, gsem1,
              ssem0, ssem1, dsem0, dsem1, psem0, psem1):
    cid = lax.axis_index("c")
    sid = lax.axis_index("s")
    wid = cid * _NTS + sid
    fb = (fb0, fb1)
    isem = (isem0, isem1)
    gsem = (gsem0, gsem1)
    ssem = (ssem0, ssem1)
    dsem = (dsem0, dsem1)
    psem = (psem0, psem1)

    ld_m = pltpu.async_copy(m_hbm, m_v, lsem)

    def _zrow(r, carry):
        for j in range(_D // 16):
            fb0[r, pl.ds(j * 16, 16)] = jnp.zeros((16,), jnp.float32)
        return carry

    lax.fori_loop(0, _CH, _zrow, 0)
    for g in range(_G16):
        pv[0, pl.ds(g * 16, 16)] = jnp.zeros((16,), jnp.float32)

    base_r = sid * _RPT
    _zchunks = [(0, _CH), (_CH, _CH), (2 * _CH, _CH), (3 * _CH, _CH),
                (4 * _CH, _CH), (5 * _CH, _RPT - 5 * _CH)]

    def _zero_range(r0, chunks):
        for off, ln in chunks:
            pltpu.sync_copy(fb0.at[pl.ds(0, ln)],
                            acc_sh.at[pl.ds(r0 + off, ln)])
            pltpu.sync_copy(pv.at[0, pl.ds(0, ln)],
                            accd_sh.at[pl.ds(r0 + off, ln)])

    _zero_range(base_r, _zchunks)

    @pl.when(sid == _NTS - 1)
    def _():
        _zero_range(_NTS * _RPT, [(0, _AR - _NTS * _RPT)])

    plsc.subcore_barrier()
    ld_m.wait()
    mvec = m_v[...]

    ebase = wid * _EPT

    def _load_idx(c, b):
        pltpu.async_copy(src_hbm.at[pl.ds(ebase + c * _CH, _CH)], srcv.at[b],
                         isem[b])
        pltpu.async_copy(dst_hbm.at[pl.ds(ebase + c * _CH, _CH)], dstv.at[b],
                         isem[b])

    def _wait_idx(b):
        pltpu.make_async_copy(src_hbm.at[pl.ds(0, _CH)], srcv.at[b],
                              isem[b]).wait()
        pltpu.make_async_copy(dst_hbm.at[pl.ds(0, _CH)], dstv.at[b],
                              isem[b]).wait()

    def _start_gather(b):
        pltpu.async_copy(xp_hbm.at[srcv.at[b]], fb[b], gsem[b])
        pltpu.async_copy(as_hbm.at[srcv.at[b]], asg.at[b], gsem[b])
        pltpu.async_copy(ad_hbm.at[dstv.at[b]], adg.at[b], gsem[b])

    def _wait_gather(b):
        pltpu.make_async_copy(xp_hbm.at[pl.ds(0, _CH)], fb[b],
                              gsem[b]).wait()
        pltpu.make_async_copy(as_hbm.at[pl.ds(0, _CH)], asg.at[b],
                              gsem[b]).wait()
        pltpu.make_async_copy(ad_hbm.at[pl.ds(0, _CH)], adg.at[b],
                              gsem[b]).wait()

    def _start_scatter(c, b):
        pltpu.async_copy(fb[b], acc_sh.at[dscv.at[b]], ssem[b], add=True)
        pltpu.async_copy(pv.at[b], accd_sh.at[dscv.at[b]], dsem[b], add=True)
        pltpu.async_copy(pv.at[b], p_out.at[pl.ds(ebase + c * _CH, _CH)],
                         psem[b])

    def _wait_scatter(b):
        pltpu.make_async_copy(acc_out.at[0, pl.ds(0, _CH)], fb[b],
                              ssem[b]).wait()
        pltpu.make_async_copy(as_hbm.at[pl.ds(0, _CH)], pv.at[b],
                              dsem[b]).wait()
        pltpu.make_async_copy(pv.at[b], p_out.at[pl.ds(0, _CH)],
                              psem[b]).wait()

    def _compute_p(b):
        for g in range(_G16):
            sl = pl.ds(g * 16, 16)
            e = asg[b, sl] + adg[b, sl]
            e = jnp.where(e < 0.0, e * jnp.float32(0.2), e)
            pv[b, sl] = jnp.exp(e - mvec)
            dscv[b, sl] = dstv[b, sl]

    def _scale(b):
        def _edge(r, c2):
            rb = jnp.broadcast_to(r, (16,)).astype(jnp.int32)
            pb = plsc.load_gather(pv.at[b], [rb])
            for q in range(_D // 16):
                sl2 = pl.ds(q * 16, 16)
                fb[b][r, sl2] = fb[b][r, sl2] * pb
            return c2

        lax.fori_loop(0, _CH, _edge, 0, unroll=4)

    def _slot(c, b, wait_prev=True, load_next2=True, start_next=True):
        bn = 1 - b
        if start_next:
            _wait_idx(bn)
            _start_gather(bn)
        if wait_prev:
            _wait_scatter(b)
        _wait_gather(b)
        _compute_p(b)
        if load_next2:
            _load_idx(c + 2, b)
        _scale(b)
        _start_scatter(c, b)

    _load_idx(jnp.int32(0), 0)
    _wait_idx(0)
    _start_gather(0)
    _load_idx(jnp.int32(1), 1)
    _slot(jnp.int32(0), 0, wait_prev=False)
    _slot(jnp.int32(1), 1, wait_prev=False)

    def _pair(j, carry):
        c = 2 * j
        _slot(c, 0)
        _slot(c + 1, 1)
        return carry

    lax.fori_loop(1, (_CHUNKS - 3) // 2, _pair, 0)

    _slot(jnp.int32(_CHUNKS - 3), 0)
    _slot(jnp.int32(_CHUNKS - 2), 1, load_next2=False)
    _slot(jnp.int32(_CHUNKS - 1), 0, load_next2=False,
          start_next=False)
    _wait_scatter(1)
    _wait_scatter(0)
    plsc.subcore_barrier()

    def _copy_range(r0, chunks, accd_dst):
        for off, ln in chunks:
            pltpu.sync_copy(acc_sh.at[pl.ds(r0 + off, ln)],
                            fb0.at[pl.ds(0, ln)])
            pltpu.sync_copy(fb0.at[pl.ds(0, ln)],
                            acc_out.at[cid, pl.ds(r0 + off, ln)])
            pltpu.sync_copy(accd_sh.at[pl.ds(r0 + off, ln)],
                            pv.at[0, pl.ds(0, ln)])
            pltpu.sync_copy(pv.at[0, pl.ds(0, ln)],
                            accd_dst.at[pl.ds(r0 + off, ln)])

    @pl.when(cid == 0)
    def _():
        _copy_range(base_r, _zchunks, accd_out0)

        @pl.when(sid == _NTS - 1)
        def _():
            _copy_range(_NTS * _RPT, [(0, _AR - _NTS * _RPT)], accd_out0)

    @pl.when(cid == 1)
    def _():
        _copy_range(base_r, _zchunks, accd_out1)

        @pl.when(sid == _NTS - 1)
        def _():
            _copy_range(_NTS * _RPT, [(0, _AR - _NTS * _RPT)], accd_out1)


def _tc_finish_body(acc_ref, accd_ref, b_ref, h_ref):
    s = acc_ref[0] + acc_ref[1]
    den = accd_ref[0] + accd_ref[1]
    rr = 1.0 / (den + jnp.float32(1e-16))
    h_ref[...] = s * rr + b_ref[...]


_tc_finish = pl.pallas_call(
    _tc_finish_body,
    out_shape=jax.ShapeDtypeStruct((_AR, _D), jnp.float32),
)


@functools.partial(
    pl.kernel,
    out_type=jax.ShapeDtypeStruct((_EP,), jnp.float32),
    mesh=_MESH,
    scratch_types=[
        pltpu.VMEM((_NPD,), jnp.float32),
        pltpu.VMEM((_NPD,), jnp.float32),
        pltpu.VMEM((_EPT,), jnp.float32),
        pltpu.VMEM((_EPT,), jnp.int32),
        pltpu.VMEM((_EPT,), jnp.float32),
        pltpu.SemaphoreType.DMA,
    ],
    compiler_params=_SC_PARAMS,
)
def _sc_alpha(p_hbm, dst_hbm, accd0_hbm, accd1_hbm, alpha_out,
              r_v, d1_v, pall_v, dstall_v, aall_v, lsem):
    cid = lax.axis_index("c")
    sid = lax.axis_index("s")
    wid = cid * _NTS + sid
    ebase = wid * _EPT
    lds = [
        pltpu.async_copy(accd0_hbm, r_v.at[pl.ds(0, _AR)], lsem),
        pltpu.async_copy(accd1_hbm, d1_v.at[pl.ds(0, _AR)], lsem),
        pltpu.async_copy(p_hbm.at[pl.ds(ebase, _EPT)], pall_v, lsem),
        pltpu.async_copy(dst_hbm.at[pl.ds(ebase, _EPT)], dstall_v, lsem),
    ]
    for h in lds:
        h.wait()

    def _rinit(i, carry):
        sl = pl.ds(i * 16, 16)
        r_v[sl] = 1.0 / (r_v[sl] + d1_v[sl] + jnp.float32(1e-16))
        return carry

    lax.fori_loop(0, _NPD // 16, _rinit, 0)

    def _grp(i, carry):
        sl = pl.ds(i * 16, 16)
        rv = plsc.load_gather(r_v, [dstall_v[sl]])
        aall_v[sl] = pall_v[sl] * rv
        return carry

    lax.fori_loop(0, _EPT // 16, _grp, 0, unroll=4)
    pltpu.sync_copy(aall_v, alpha_out.at[pl.ds(ebase, _EPT)])


def kernel(x, edge_index, W, att_src, att_dst, bias):
    xp, a_s, a_d, m = _tc_prep(
        x, W, att_src.reshape(1, _D), att_dst.reshape(1, _D)
    )
    a_s = a_s.reshape(_N)
    a_d = jnp.concatenate([a_d.reshape(_N), jnp.zeros((_NPD - _N,), jnp.float32)])
    m16 = m.reshape(128)[:16]

    loop = jnp.arange(_N, dtype=jnp.int32)
    npad_e = _EP - _EN
    src_full = jnp.concatenate(
        [edge_index[0], loop, jnp.zeros((npad_e,), jnp.int32)])
    dst_full = jnp.concatenate(
        [edge_index[1], loop, jnp.full((npad_e,), _N, jnp.int32)])

    acc, accd0, accd1, p = _sc_edges(xp, src_full, dst_full, a_s, a_d, m16)
    accd = jnp.stack([accd0, accd1]).reshape(_NSC, _AR, 1)
    h_full = _tc_finish(acc, accd, bias.reshape(1, _D))
    alpha = _sc_alpha(p, dst_full, accd0, accd1)
    return h_full[:_N], alpha[:_EN].reshape(_EN, 1)
